# baseline (speedup 1.0000x reference)
import jax
import jax.numpy as jnp
from jax import lax
from jax.experimental import pallas as pl
from jax.experimental.pallas import tpu as pltpu
from jax.experimental.pallas import tpu_sc as plsc

B = 16384
H = 64
NC, NS = 2, 16
NW = NC * NS
ROWS_PER_W = B // NW
R = 32
NCHUNK = ROWS_PER_W // R

LN2 = 0.6931471805599453
SQRT2 = 1.4142135623730951
C_BING = 0.6931471805599453 + 1.5 * 1.1447298858494002
LOG2PI3 = 3.0 * 1.8378770664093453


def _ln(x):
    bits = plsc.bitcast(x, jnp.int32)
    e = (bits >> 23) - 127
    m = plsc.bitcast((bits & 0x007FFFFF) | 0x3F800000, jnp.float32)
    big = m > SQRT2
    m = jnp.where(big, m * 0.5, m)
    ef = e.astype(jnp.float32) + jnp.where(big, 1.0, 0.0)
    t = (m - 1.0) / (m + 1.0)
    t2 = t * t
    poly = t * (2.0 + t2 * (0.66666667 + t2 * (0.4 + t2 * (0.28571429 + t2 * 0.22222222))))
    return ef * LN2 + poly


def _rsqrt(x):
    bits = plsc.bitcast(x, jnp.int32)
    y = plsc.bitcast(0x5F3759DF - (bits >> 1), jnp.float32)
    for _ in range(3):
        y = y * (1.5 - 0.5 * x * y * y)
    return y


def _body(pq_h, pl_h, w_h, gq_h, px_h, pv_h, gt_h, km_h, out_h, *scr):
    bufs = (scr[0:7], scr[7:14])
    kmv, outv, idxv, sem0, sem1 = scr[14:19]
    sems = (sem0, sem1)

    cid = lax.axis_index("c")
    sid = lax.axis_index("s")
    wid = sid * NC + cid
    row0 = wid * ROWS_PER_W

    pltpu.sync_copy(km_h, kmv)
    km = kmv[...]

    def dma_list(g, b):
        base = row0 + g * R
        qv, lv, wv, gqv, xv, vv, gtv = bufs[b]
        sem = sems[b]
        return (
            (pq_h.at[pl.ds(base * 256, R * 256)], qv, sem),
            (pl_h.at[pl.ds(base * 192, R * 192)], lv, sem),
            (w_h.at[pl.ds(base * 64, R * 64)], wv, sem),
            (gq_h.at[pl.ds(base * 4, R * 4)], gqv, sem),
            (px_h.at[pl.ds(base * 192, R * 192)], xv, sem),
            (pv_h.at[pl.ds(base * 192, R * 192)], vv, sem),
            (gt_h.at[pl.ds(base * 4, R * 4)], gtv, sem),
        )

    def start_chunk(g, b):
        for src, dst, sem in dma_list(g, b):
            pltpu.async_copy(src, dst, sem)

    def wait_chunk(g, b):
        for src, dst, sem in dma_list(g, b):
            pltpu.make_async_copy(src, dst, sem).wait()

    iota = lax.broadcasted_iota(jnp.int32, (16,), 0)
    iota4 = iota * 4
    iota3 = iota * 3
    lane_lt8 = iota < 8

    def splat(ref, i):
        return plsc.load_gather(ref, [jnp.broadcast_to(i, (16,))])

    def select_rows(r, carry, b):
        qv, lv, wv, gqv, xv, vv, gtv = bufs[b]
        qb = r * 256
        xb = r * 192

        tk = None
        tv = None
        for j in range(4):
            acc = None
            for c in range(4):
                qc = plsc.load_gather(qv, [iota4 + (qb + 64 * j + c)])
                d = jnp.abs(qc - splat(gqv, r * 4 + c))
                acc = d if acc is None else acc + d
            for c in range(3):
                xc = plsc.load_gather(xv, [iota3 + (xb + 48 * j + c)])
                acc = acc + jnp.abs(xc - splat(gtv, r * 4 + c))
            sk, sv = plsc.sort_key_val(acc, iota + 16 * j)
            if tk is None:
                tk, tv = sk, sv
            else:
                ck = jnp.where(lane_lt8, tk, jnp.flip(sk))
                cv = jnp.where(lane_lt8, tv, jnp.flip(sv))
                tk, tv = plsc.sort_key_val(ck, cv)
        idxv[pl.ds(r * 16, 16)] = tv
        return carry

    def prob_rows(r, accs, b):
        qv, lv, wv, gqv, xv, vv, gtv = bufs[b]
        accp, accg, accb, accw = accs
        qb = r * 256
        xb = r * 192
        wb = r * 64
        hsel = idxv[pl.ds(r * 16, 16)]

        qs = [plsc.load_gather(qv, [qb + hsel * 4 + c]) for c in range(4)]
        s = qs[0] * qs[0] + qs[1] * qs[1] + qs[2] * qs[2] + qs[3] * qs[3]
        rn = _rsqrt(s)
        a_ = qs[0] * rn
        b_ = qs[1] * rn
        c_ = qs[2] * rn
        d_ = qs[3] * rn
        g0 = splat(gqv, r * 4 + 0)
        g1 = splat(gqv, r * 4 + 1)
        g2 = splat(gqv, r * 4 + 2)
        g3 = splat(gqv, r * 4 + 3)
        t1 = g1 * a_ - g0 * b_ + g3 * c_ - g2 * d_
        t2 = g2 * a_ - g3 * b_ - g0 * c_ + g1 * d_
        t3 = g3 * a_ + g2 * b_ - g1 * c_ - g0 * d_
        p0 = jnp.abs(plsc.load_gather(lv, [xb + hsel * 3])) + 1e-6
        p1 = jnp.abs(plsc.load_gather(lv, [xb + hsel * 3 + 1])) + 1e-6
        p2 = jnp.abs(plsc.load_gather(lv, [xb + hsel * 3 + 2])) + 1e-6
        quad = p0 * t1 * t1 + p1 * t2 * t2 + p2 * t3 * t3
        log_norm = C_BING - 0.5 * _ln(p0 * p1 * p2)
        accp = accp + (quad + log_norm) * km

        xq = None
        ldet = None
        for c in range(3):
            vc = plsc.load_gather(vv, [xb + hsel * 3 + c]) + 1e-8
            dxc = plsc.load_gather(xv, [xb + hsel * 3 + c]) - splat(gtv, r * 4 + c)
            term = dxc * dxc / vc
            xq = term if xq is None else xq + term
            ldet = vc if ldet is None else ldet * vc
        accg = accg + (xq + LOG2PI3 + _ln(ldet)) * km

        accw = accw + plsc.load_gather(wv, [wb + hsel]) * km
        for j in range(4):
            wj = wv[pl.ds(wb + 16 * j, 16)]
            accb = accb + (0.5 * (wj + jnp.abs(wj))
                           + _ln(1.0 + jnp.exp(-jnp.abs(wj))))
        return (accp, accg, accb, accw)

    zero = jnp.zeros((16,), jnp.float32)
    accs = (zero, zero, zero, zero)
    start_chunk(0, 0)

    def pair_body(i, accs):
        for b in (0, 1):
            g = i * 2 + b
            wait_chunk(g, b)

            @pl.when(g + 1 < NCHUNK)
            def _():
                start_chunk(g + 1, 1 - b)

            lax.fori_loop(0, R, lambda r, a: select_rows(r, a, b), 0)
            accs = lax.fori_loop(0, R, lambda r, a: prob_rows(r, a, b), accs)
        return accs

    accp, accg, accb, accw = lax.fori_loop(0, NCHUNK // 2, pair_body, accs)

    outv[pl.ds(0, 16)] = accp
    outv[pl.ds(16, 16)] = accg
    outv[pl.ds(32, 16)] = accb
    outv[pl.ds(48, 16)] = accw
    pltpu.sync_copy(outv, out_h.at[pl.ds(wid * 64, 64)])


def _make_call():
    mesh = plsc.VectorSubcoreMesh(core_axis_name="c", subcore_axis_name="s",
                                  num_cores=NC, num_subcores=NS)
    scratch = []
    for _ in range(2):
        scratch += [
            pltpu.VMEM((R * 256,), jnp.float32),
            pltpu.VMEM((R * 192,), jnp.float32),
            pltpu.VMEM((R * 64,), jnp.float32),
            pltpu.VMEM((R * 4,), jnp.float32),
            pltpu.VMEM((R * 192,), jnp.float32),
            pltpu.VMEM((R * 192,), jnp.float32),
            pltpu.VMEM((R * 4,), jnp.float32),
        ]
    scratch += [
        pltpu.VMEM((16,), jnp.float32),
        pltpu.VMEM((64,), jnp.float32),
        pltpu.VMEM((R * 16,), jnp.int32),
        pltpu.SemaphoreType.DMA,
        pltpu.SemaphoreType.DMA,
    ]
    return pl.kernel(
        _body,
        out_type=jax.ShapeDtypeStruct((NW * 64,), jnp.float32),
        mesh=mesh,
        scratch_types=scratch,
        compiler_params=pltpu.CompilerParams(needs_layout_passes=False),
    )


_sc_call = _make_call()


def kernel(pred_q, pred_l, weights, gt_q, pred_x, pred_var, gt_t, k):
    km = (jnp.arange(16) < jnp.minimum(jnp.asarray(k, jnp.int32), 8)).astype(jnp.float32)
    out = _sc_call(
        pred_q.reshape(-1), pred_l.reshape(-1), weights.reshape(-1),
        gt_q.reshape(-1), pred_x.reshape(-1), pred_var.reshape(-1),
        jnp.pad(gt_t, ((0, 0), (0, 1))).reshape(-1), km,
    )
    s = jnp.sum(out.reshape(NW, 4, 16), axis=(0, 2))
    loss = s[0] / B
    gloss = 0.5 * s[1] / B
    weight_loss = (s[2] - s[3]) / (B * H)
    return (loss, weight_loss, gloss)


# speedup vs baseline: 25.2933x; 25.2933x over previous
import jax
import jax.numpy as jnp
from jax import lax
from jax.experimental import pallas as pl
from jax.experimental.pallas import tpu as pltpu
from jax.experimental.pallas import tpu_sc as plsc

B = 16384
H = 64
NC, NS = 2, 16
NW = NC * NS
NBT = B // 128
TPW = NBT // NW
R = 128

LN2 = 0.6931471805599453
SQRT2 = 1.4142135623730951
C_BING = 0.6931471805599453 + 1.5 * 1.1447298858494002
LOG2PI3 = 3.0 * 1.8378770664093453


def _ln(x):
    bits = plsc.bitcast(x, jnp.int32)
    e = (bits >> 23) - 127
    m = plsc.bitcast((bits & 0x007FFFFF) | 0x3F800000, jnp.float32)
    big = m > SQRT2
    m = jnp.where(big, m * 0.5, m)
    ef = e.astype(jnp.float32) + jnp.where(big, 1.0, 0.0)
    t = (m - 1.0) / (m + 1.0)
    t2 = t * t
    poly = t * (2.0 + t2 * (0.66666667 + t2 * (0.4 + t2 * (0.28571429 + t2 * 0.22222222))))
    return ef * LN2 + poly


def _rsqrt(x):
    bits = plsc.bitcast(x, jnp.int32)
    y = plsc.bitcast(0x5F3759DF - (bits >> 1), jnp.float32)
    for _ in range(3):
        y = y * (1.5 - 0.5 * x * y * y)
    return y


def _body(q_h, l_h, w_h, gq_h, x_h, v_h, gt_h, km_h, out_h, *scr):
    qv, lv, xv, vv, wv, gqv, gtv, kmv, outv, idxv, sem = scr

    cid = lax.axis_index("c")
    sid = lax.axis_index("s")
    wid = sid * NC + cid

    pltpu.sync_copy(km_h, kmv)
    km = kmv[...]

    iota = lax.broadcasted_iota(jnp.int32, (16,), 0)
    lane_lt8 = iota < 8
    zero16 = jnp.zeros((16,), jnp.int32)

    def splat(i):
        return zero16 + i

    def dma_list(bt):
        return (
            (q_h.at[:, bt, :, :], qv),
            (l_h.at[:, :, bt, :, :], lv),
            (x_h.at[:, :, bt, :, :], xv),
            (v_h.at[:, :, bt, :, :], vv),
            (w_h.at[:, bt, :, :], wv),
            (gq_h.at[bt, :, :], gqv),
            (gt_h.at[bt, :, :], gtv),
        )

    def select_rows(r, carry):
        rs = splat(r)
        tk = None
        tv = None
        for j in range(4):
            hvec = iota + 16 * j
            htv = hvec >> 3
            hsv = iota & 7
            acc = None
            for c in range(4):
                qc = plsc.load_gather(qv, [hvec, splat(c), rs])
                gc = plsc.load_gather(gqv, [splat(c), rs])
                d = jnp.abs(qc - gc)
                acc = d if acc is None else acc + d
            for c in range(3):
                xc = plsc.load_gather(xv, [splat(c), htv, hsv, rs])
                tc = plsc.load_gather(gtv, [splat(c), rs])
                acc = acc + jnp.abs(xc - tc)
            sk, sv = plsc.sort_key_val(acc, hvec)
            if tk is None:
                tk, tv = sk, sv
            else:
                ck = jnp.where(lane_lt8, tk, jnp.flip(sk))
                cv = jnp.where(lane_lt8, tv, jnp.flip(sv))
                tk, tv = plsc.sort_key_val(ck, cv)
        idxv[pl.ds(r * 16, 16)] = tv
        return carry

    def prob_rows(r, accs):
        accp, accg, accb, accw = accs
        rs = splat(r)
        hsel = idxv[pl.ds(r * 16, 16)]
        hst = hsel >> 3
        hss = hsel & 7

        qs = [plsc.load_gather(qv, [hsel, splat(c), rs]) for c in range(4)]
        s = qs[0] * qs[0] + qs[1] * qs[1] + qs[2] * qs[2] + qs[3] * qs[3]
        rn = _rsqrt(s)
        a_ = qs[0] * rn
        b_ = qs[1] * rn
        c_ = qs[2] * rn
        d_ = qs[3] * rn
        g0 = plsc.load_gather(gqv, [splat(0), rs])
        g1 = plsc.load_gather(gqv, [splat(1), rs])
        g2 = plsc.load_gather(gqv, [splat(2), rs])
        g3 = plsc.load_gather(gqv, [splat(3), rs])
        t1 = g1 * a_ - g0 * b_ + g3 * c_ - g2 * d_
        t2 = g2 * a_ - g3 * b_ - g0 * c_ + g1 * d_
        t3 = g3 * a_ + g2 * b_ - g1 * c_ - g0 * d_
        p0 = jnp.abs(plsc.load_gather(lv, [splat(0), hst, hss, rs])) + 1e-6
        p1 = jnp.abs(plsc.load_gather(lv, [splat(1), hst, hss, rs])) + 1e-6
        p2 = jnp.abs(plsc.load_gather(lv, [splat(2), hst, hss, rs])) + 1e-6
        quad = p0 * t1 * t1 + p1 * t2 * t2 + p2 * t3 * t3
        log_norm = C_BING - 0.5 * _ln(p0 * p1 * p2)
        accp = accp + (quad + log_norm) * km

        xq = None
        ldet = None
        for c in range(3):
            vc = plsc.load_gather(vv, [splat(c), hst, hss, rs]) + 1e-8
            tc = plsc.load_gather(gtv, [splat(c), rs])
            dxc = plsc.load_gather(xv, [splat(c), hst, hss, rs]) - tc
            term = dxc * dxc / vc
            xq = term if xq is None else xq + term
            ldet = vc if ldet is None else ldet * vc
        accg = accg + (xq + LOG2PI3 + _ln(ldet)) * km

        accw = accw + plsc.load_gather(wv, [hst, hss, rs]) * km
        for j in range(4):
            wj = plsc.load_gather(wv, [(iota + 16 * j) >> 3, iota & 7, rs])
            accb = accb + (0.5 * (wj + jnp.abs(wj))
                           + _ln(1.0 + jnp.exp(-jnp.abs(wj))))
        return (accp, accg, accb, accw)

    zero = jnp.zeros((16,), jnp.float32)
    accs = (zero, zero, zero, zero)

    def chunk_body(ct, accs):
        bt = wid * TPW + ct
        copies = [pltpu.make_async_copy(src, dst, sem)
                  for src, dst in dma_list(bt)]
        for cp in copies:
            cp.start()
        for cp in copies:
            cp.wait()
        lax.fori_loop(0, R, select_rows, 0)
        return lax.fori_loop(0, R, prob_rows, accs)

    accp, accg, accb, accw = lax.fori_loop(0, TPW, chunk_body, accs)

    outv[pl.ds(0, 16)] = accp
    outv[pl.ds(16, 16)] = accg
    outv[pl.ds(32, 16)] = accb
    outv[pl.ds(48, 16)] = accw
    pltpu.sync_copy(outv, out_h.at[pl.ds(wid * 64, 64)])


def _make_call():
    mesh = plsc.VectorSubcoreMesh(core_axis_name="c", subcore_axis_name="s",
                                  num_cores=NC, num_subcores=NS)
    scratch = [
        pltpu.VMEM((64, 4, 128), jnp.float32),
        pltpu.VMEM((3, 8, 8, 128), jnp.float32),
        pltpu.VMEM((3, 8, 8, 128), jnp.float32),
        pltpu.VMEM((3, 8, 8, 128), jnp.float32),
        pltpu.VMEM((8, 8, 128), jnp.float32),
        pltpu.VMEM((4, 128), jnp.float32),
        pltpu.VMEM((4, 128), jnp.float32),
        pltpu.VMEM((16,), jnp.float32),
        pltpu.VMEM((64,), jnp.float32),
        pltpu.VMEM((R * 16,), jnp.int32),
        pltpu.SemaphoreType.DMA,
    ]
    return pl.kernel(
        _body,
        out_type=jax.ShapeDtypeStruct((NW * 64,), jnp.float32),
        mesh=mesh,
        scratch_types=scratch,
        compiler_params=pltpu.CompilerParams(needs_layout_passes=False),
    )


_sc_call = _make_call()


def kernel(pred_q, pred_l, weights, gt_q, pred_x, pred_var, gt_t, k):
    km = (jnp.arange(16) < jnp.minimum(jnp.asarray(k, jnp.int32), 8)).astype(jnp.float32)
    qP = pred_q.reshape(128, 128, 64, 4).transpose(2, 0, 3, 1)
    lP = pred_l.reshape(128, 128, 8, 8, 3).transpose(4, 2, 0, 3, 1)
    xP = pred_x.reshape(128, 128, 8, 8, 3).transpose(4, 2, 0, 3, 1)
    vP = pred_var.reshape(128, 128, 8, 8, 3).transpose(4, 2, 0, 3, 1)
    wP = weights.reshape(128, 128, 8, 8).transpose(2, 0, 3, 1)
    gqP = gt_q.reshape(128, 128, 4).transpose(0, 2, 1)
    gtP = jnp.pad(gt_t, ((0, 0), (0, 1))).reshape(128, 128, 4).transpose(0, 2, 1)
    out = _sc_call(qP, lP, wP, gqP, xP, vP, gtP, km)
    s = jnp.sum(out.reshape(NW, 4, 16), axis=(0, 2))
    loss = s[0] / B
    gloss = 0.5 * s[1] / B
    weight_loss = (s[2] - s[3]) / (B * H)
    return (loss, weight_loss, gloss)


# speedup vs baseline: 26.2185x; 1.0366x over previous
import jax
import jax.numpy as jnp
from jax import lax
from jax.experimental import pallas as pl
from jax.experimental.pallas import tpu as pltpu
from jax.experimental.pallas import tpu_sc as plsc

B = 16384
H = 64
NC, NS = 2, 16
NW = NC * NS
NBT = B // 128
TPW = NBT // NW
R = 128

LN2 = 0.6931471805599453
SQRT2 = 1.4142135623730951
C_BING = 0.6931471805599453 + 1.5 * 1.1447298858494002
LOG2PI3 = 3.0 * 1.8378770664093453


def _ln(x):
    bits = plsc.bitcast(x, jnp.int32)
    e = (bits >> 23) - 127
    m = plsc.bitcast((bits & 0x007FFFFF) | 0x3F800000, jnp.float32)
    big = m > SQRT2
    m = jnp.where(big, m * 0.5, m)
    ef = e.astype(jnp.float32) + jnp.where(big, 1.0, 0.0)
    t = (m - 1.0) / (m + 1.0)
    t2 = t * t
    poly = t * (2.0 + t2 * (0.66666667 + t2 * (0.4 + t2 * (0.28571429 + t2 * 0.22222222))))
    return ef * LN2 + poly


def _rsqrt(x):
    bits = plsc.bitcast(x, jnp.int32)
    y = plsc.bitcast(0x5F3759DF - (bits >> 1), jnp.float32)
    for _ in range(3):
        y = y * (1.5 - 0.5 * x * y * y)
    return y


def _body(q_h, l_h, w_h, gq_h, x_h, v_h, gt_h, km_h, out_h, *scr):
    qv, lv, xv, vv, wv, gqv, gtv, kmv, outv, idxv, sem = scr

    cid = lax.axis_index("c")
    sid = lax.axis_index("s")
    wid = sid * NC + cid

    pltpu.sync_copy(km_h, kmv)
    km = kmv[...]

    iota = lax.broadcasted_iota(jnp.int32, (16,), 0)
    lane_lt8 = iota < 8
    zero16 = jnp.zeros((16,), jnp.int32)

    def splat(i):
        return zero16 + i

    def dma_list(bt):
        return (
            (q_h.at[:, bt, :, :], qv),
            (l_h.at[:, :, bt, :, :], lv),
            (x_h.at[:, :, bt, :, :], xv),
            (v_h.at[:, :, bt, :, :], vv),
            (w_h.at[:, bt, :, :], wv),
            (gq_h.at[bt, :, :], gqv),
            (gt_h.at[bt, :, :], gtv),
        )

    def select_rows(r, carry):
        rs = splat(r)
        gs = [plsc.load_gather(gqv, [splat(c), rs]) for c in range(4)]
        ts = [plsc.load_gather(gtv, [splat(c), rs]) for c in range(3)]
        tk = None
        tv = None
        for j in range(4):
            hvec = iota + 16 * j
            htv = hvec >> 3
            hsv = iota & 7
            acc = None
            for c in range(4):
                qc = plsc.load_gather(qv, [hvec, splat(c), rs])
                d = jnp.abs(qc - gs[c])
                acc = d if acc is None else acc + d
            for c in range(3):
                xc = plsc.load_gather(xv, [splat(c), htv, hsv, rs])
                acc = acc + jnp.abs(xc - ts[c])
            sk, sv = plsc.sort_key_val(acc, hvec)
            if tk is None:
                tk, tv = sk, sv
            else:
                ck = jnp.where(lane_lt8, tk, jnp.flip(sk))
                cv = jnp.where(lane_lt8, tv, jnp.flip(sv))
                tk, tv = plsc.sort_key_val(ck, cv)
        idxv[pl.ds(r * 16, 16)] = tv
        return carry

    def prob_rows(r, accs):
        accp, accg, accb, accw = accs
        rs = splat(r)
        hsel = idxv[pl.ds(r * 16, 16)]
        hst = hsel >> 3
        hss = hsel & 7

        qs = [plsc.load_gather(qv, [hsel, splat(c), rs]) for c in range(4)]
        s = qs[0] * qs[0] + qs[1] * qs[1] + qs[2] * qs[2] + qs[3] * qs[3]
        rn = _rsqrt(s)
        a_ = qs[0] * rn
        b_ = qs[1] * rn
        c_ = qs[2] * rn
        d_ = qs[3] * rn
        g0 = plsc.load_gather(gqv, [splat(0), rs])
        g1 = plsc.load_gather(gqv, [splat(1), rs])
        g2 = plsc.load_gather(gqv, [splat(2), rs])
        g3 = plsc.load_gather(gqv, [splat(3), rs])
        t1 = g1 * a_ - g0 * b_ + g3 * c_ - g2 * d_
        t2 = g2 * a_ - g3 * b_ - g0 * c_ + g1 * d_
        t3 = g3 * a_ + g2 * b_ - g1 * c_ - g0 * d_
        p0 = jnp.abs(plsc.load_gather(lv, [splat(0), hst, hss, rs])) + 1e-6
        p1 = jnp.abs(plsc.load_gather(lv, [splat(1), hst, hss, rs])) + 1e-6
        p2 = jnp.abs(plsc.load_gather(lv, [splat(2), hst, hss, rs])) + 1e-6
        quad = p0 * t1 * t1 + p1 * t2 * t2 + p2 * t3 * t3
        log_norm = C_BING - 0.5 * _ln(p0 * p1 * p2)
        accp = accp + (quad + log_norm) * km

        xq = None
        ldet = None
        for c in range(3):
            vc = plsc.load_gather(vv, [splat(c), hst, hss, rs]) + 1e-8
            tc = plsc.load_gather(gtv, [splat(c), rs])
            dxc = plsc.load_gather(xv, [splat(c), hst, hss, rs]) - tc
            term = dxc * dxc / vc
            xq = term if xq is None else xq + term
            ldet = vc if ldet is None else ldet * vc
        accg = accg + (xq + LOG2PI3 + _ln(ldet)) * km

        accw = accw + plsc.load_gather(wv, [hst, hss, rs]) * km
        for j in range(4):
            wj = plsc.load_gather(wv, [(iota + 16 * j) >> 3, iota & 7, rs])
            accb = accb + (0.5 * (wj + jnp.abs(wj))
                           + _ln(1.0 + jnp.exp(-jnp.abs(wj))))
        return (accp, accg, accb, accw)

    zero = jnp.zeros((16,), jnp.float32)
    accs = (zero, zero, zero, zero)

    def chunk_body(ct, accs):
        bt = wid * TPW + ct
        copies = [pltpu.make_async_copy(src, dst, sem)
                  for src, dst in dma_list(bt)]
        for cp in copies:
            cp.start()
        for cp in copies:
            cp.wait()
        lax.fori_loop(0, R, select_rows, 0, unroll=2)
        return lax.fori_loop(0, R, prob_rows, accs, unroll=2)

    accp, accg, accb, accw = lax.fori_loop(0, TPW, chunk_body, accs)

    outv[pl.ds(0, 16)] = accp
    outv[pl.ds(16, 16)] = accg
    outv[pl.ds(32, 16)] = accb
    outv[pl.ds(48, 16)] = accw
    pltpu.sync_copy(outv, out_h.at[pl.ds(wid * 64, 64)])


def _make_call():
    mesh = plsc.VectorSubcoreMesh(core_axis_name="c", subcore_axis_name="s",
                                  num_cores=NC, num_subcores=NS)
    scratch = [
        pltpu.VMEM((64, 4, 128), jnp.float32),
        pltpu.VMEM((3, 8, 8, 128), jnp.float32),
        pltpu.VMEM((3, 8, 8, 128), jnp.float32),
        pltpu.VMEM((3, 8, 8, 128), jnp.float32),
        pltpu.VMEM((8, 8, 128), jnp.float32),
        pltpu.VMEM((4, 128), jnp.float32),
        pltpu.VMEM((4, 128), jnp.float32),
        pltpu.VMEM((16,), jnp.float32),
        pltpu.VMEM((64,), jnp.float32),
        pltpu.VMEM((R * 16,), jnp.int32),
        pltpu.SemaphoreType.DMA,
    ]
    return pl.kernel(
        _body,
        out_type=jax.ShapeDtypeStruct((NW * 64,), jnp.float32),
        mesh=mesh,
        scratch_types=scratch,
        compiler_params=pltpu.CompilerParams(needs_layout_passes=False),
    )


_sc_call = _make_call()


def kernel(pred_q, pred_l, weights, gt_q, pred_x, pred_var, gt_t, k):
    km = (jnp.arange(16) < jnp.minimum(jnp.asarray(k, jnp.int32), 8)).astype(jnp.float32)
    qP = pred_q.reshape(128, 128, 64, 4).transpose(2, 0, 3, 1)
    lP = pred_l.reshape(128, 128, 8, 8, 3).transpose(4, 2, 0, 3, 1)
    xP = pred_x.reshape(128, 128, 8, 8, 3).transpose(4, 2, 0, 3, 1)
    vP = pred_var.reshape(128, 128, 8, 8, 3).transpose(4, 2, 0, 3, 1)
    wP = weights.reshape(128, 128, 8, 8).transpose(2, 0, 3, 1)
    gqP = gt_q.reshape(128, 128, 4).transpose(0, 2, 1)
    gtP = jnp.pad(gt_t, ((0, 0), (0, 1))).reshape(128, 128, 4).transpose(0, 2, 1)
    out = _sc_call(qP, lP, wP, gqP, xP, vP, gtP, km)
    s = jnp.sum(out.reshape(NW, 4, 16), axis=(0, 2))
    loss = s[0] / B
    gloss = 0.5 * s[1] / B
    weight_loss = (s[2] - s[3]) / (B * H)
    return (loss, weight_loss, gloss)


# speedup vs baseline: 55.0130x; 2.0983x over previous
import jax
import jax.numpy as jnp
from jax import lax
from jax.experimental import pallas as pl
from jax.experimental.pallas import tpu as pltpu
from jax.experimental.pallas import tpu_sc as plsc

B = 16384
H = 64
NC, NS = 2, 16
NW = NC * NS
TPW = (B // 128) // NW
R = 128
PITCH = 129

LN2 = 0.6931471805599453
SQRT2 = 1.4142135623730951
C_BING = 0.6931471805599453 + 1.5 * 1.1447298858494002
LOG2PI3 = 3.0 * 1.8378770664093453


def _ln(x):
    bits = plsc.bitcast(x, jnp.int32)
    e = (bits >> 23) - 127
    m = plsc.bitcast((bits & 0x007FFFFF) | 0x3F800000, jnp.float32)
    big = m > SQRT2
    m = jnp.where(big, m * 0.5, m)
    ef = e.astype(jnp.float32) + jnp.where(big, 1.0, 0.0)
    t = (m - 1.0) / (m + 1.0)
    t2 = t * t
    poly = t * (2.0 + t2 * (0.66666667 + t2 * (0.4 + t2 * (0.28571429 + t2 * 0.22222222))))
    return ef * LN2 + poly


def _rsqrt(x):
    bits = plsc.bitcast(x, jnp.int32)
    y = plsc.bitcast(0x5F3759DF - (bits >> 1), jnp.float32)
    for _ in range(3):
        y = y * (1.5 - 0.5 * x * y * y)
    return y


def _body(q_h, l_h, w_h, gq_h, x_h, v_h, gt_h, km_h, out_h, *scr):
    qv, lv, xv, vv, wv, gqv, gtv, kmv, outv, lbuf, selv, semA, semB = scr

    cid = lax.axis_index("c")
    sid = lax.axis_index("s")
    wid = sid * NC + cid

    pltpu.sync_copy(km_h, kmv)
    km = kmv[...]

    iota = lax.broadcasted_iota(jnp.int32, (16,), 0)
    iotap = iota * PITCH
    lane_lt8 = iota < 8
    zero16 = jnp.zeros((16,), jnp.int32)

    def splat(i):
        return zero16 + i

    def phase_a(m, carry):
        rv = m * 16 + iota
        gq = [plsc.load_gather(gqv, [splat(c), rv]) for c in range(4)]
        gt = [plsc.load_gather(gtv, [splat(c), rv]) for c in range(3)]
        for h in range(H):
            ht, hs = h >> 3, h & 7
            acc = None
            for c in range(4):
                qc = plsc.load_gather(qv, [splat(h), splat(c), rv])
                d = jnp.abs(qc - gq[c])
                acc = d if acc is None else acc + d
            for c in range(3):
                xc = plsc.load_gather(xv, [splat(c), splat(ht), splat(hs), rv])
                acc = acc + jnp.abs(xc - gt[c])
            plsc.store_scatter(lbuf, [rv + h * PITCH], acc)
        return carry

    def phase_b(r, carry):
        tk = None
        tv = None
        for j in range(4):
            keys = plsc.load_gather(lbuf, [iotap + (j * 16 * PITCH + r)])
            sk, sv = plsc.sort_key_val(keys, iota + 16 * j)
            if tk is None:
                tk, tv = sk, sv
            else:
                ck = jnp.where(lane_lt8, tk, jnp.flip(sk))
                cv = jnp.where(lane_lt8, tv, jnp.flip(sv))
                tk, tv = plsc.sort_key_val(ck, cv)
        plsc.store_scatter(selv, [iotap + r], tv, mask=lane_lt8)
        return carry

    def phase_c(m, accs):
        accp, accg, accw = accs
        rv = m * 16 + iota
        gq = [plsc.load_gather(gqv, [splat(c), rv]) for c in range(4)]
        gt = [plsc.load_gather(gtv, [splat(c), rv]) for c in range(3)]
        for s in range(8):
            kms = km[s]
            hsel = plsc.load_gather(selv, [rv + s * PITCH])
            hst = hsel >> 3
            hss = hsel & 7

            qs = [plsc.load_gather(qv, [hsel, splat(c), rv]) for c in range(4)]
            sq = qs[0] * qs[0] + qs[1] * qs[1] + qs[2] * qs[2] + qs[3] * qs[3]
            rn = _rsqrt(sq)
            a_ = qs[0] * rn
            b_ = qs[1] * rn
            c_ = qs[2] * rn
            d_ = qs[3] * rn
            t1 = gq[1] * a_ - gq[0] * b_ + gq[3] * c_ - gq[2] * d_
            t2 = gq[2] * a_ - gq[3] * b_ - gq[0] * c_ + gq[1] * d_
            t3 = gq[3] * a_ + gq[2] * b_ - gq[1] * c_ - gq[0] * d_
            p0 = jnp.abs(plsc.load_gather(lv, [splat(0), hst, hss, rv])) + 1e-6
            p1 = jnp.abs(plsc.load_gather(lv, [splat(1), hst, hss, rv])) + 1e-6
            p2 = jnp.abs(plsc.load_gather(lv, [splat(2), hst, hss, rv])) + 1e-6
            quad = p0 * t1 * t1 + p1 * t2 * t2 + p2 * t3 * t3
            log_norm = C_BING - 0.5 * _ln(p0 * p1 * p2)
            accp = accp + (quad + log_norm) * kms

            xq = None
            ldet = None
            for c in range(3):
                vc = plsc.load_gather(vv, [splat(c), hst, hss, rv]) + 1e-8
                dxc = plsc.load_gather(xv, [splat(c), hst, hss, rv]) - gt[c]
                term = dxc * dxc / vc
                xq = term if xq is None else xq + term
                ldet = vc if ldet is None else ldet * vc
            accg = accg + (xq + LOG2PI3 + _ln(ldet)) * kms

            accw = accw + plsc.load_gather(wv, [hst, hss, rv]) * kms
        return (accp, accg, accw)

    def phase_w(h, accb):
        ht, hs = h >> 3, h & 7
        for m in range(8):
            rv = m * 16 + iota
            wj = plsc.load_gather(wv, [splat(ht), splat(hs), rv])
            accb = accb + (0.5 * (wj + jnp.abs(wj))
                           + _ln(1.0 + jnp.exp(-jnp.abs(wj))))
        return accb

    zero = jnp.zeros((16,), jnp.float32)

    def chunk_body(ct, accs):
        accp, accg, accb, accw = accs
        bt = wid * TPW + ct
        grp_a = [pltpu.make_async_copy(q_h.at[:, bt, :, :], qv, semA),
                 pltpu.make_async_copy(x_h.at[:, :, bt, :, :], xv, semA),
                 pltpu.make_async_copy(gq_h.at[bt, :, :], gqv, semA),
                 pltpu.make_async_copy(gt_h.at[bt, :, :], gtv, semA)]
        grp_b = [pltpu.make_async_copy(l_h.at[:, :, bt, :, :], lv, semB),
                 pltpu.make_async_copy(v_h.at[:, :, bt, :, :], vv, semB),
                 pltpu.make_async_copy(w_h.at[:, bt, :, :], wv, semB)]
        for cp in grp_a + grp_b:
            cp.start()
        for cp in grp_a:
            cp.wait()
        lax.fori_loop(0, 8, phase_a, 0)
        lax.fori_loop(0, R, phase_b, 0, unroll=2)
        for cp in grp_b:
            cp.wait()
        accp, accg, accw = lax.fori_loop(0, 8, phase_c, (accp, accg, accw))
        accb = lax.fori_loop(0, H, phase_w, accb, unroll=2)
        return (accp, accg, accb, accw)

    accp, accg, accb, accw = lax.fori_loop(
        0, TPW, chunk_body, (zero, zero, zero, zero))

    outv[pl.ds(0, 16)] = accp
    outv[pl.ds(16, 16)] = accg
    outv[pl.ds(32, 16)] = accb
    outv[pl.ds(48, 16)] = accw
    pltpu.sync_copy(outv, out_h.at[pl.ds(wid * 64, 64)])


def _make_call():
    mesh = plsc.VectorSubcoreMesh(core_axis_name="c", subcore_axis_name="s",
                                  num_cores=NC, num_subcores=NS)
    scratch = [
        pltpu.VMEM((64, 4, 128), jnp.float32),
        pltpu.VMEM((3, 8, 8, 128), jnp.float32),
        pltpu.VMEM((3, 8, 8, 128), jnp.float32),
        pltpu.VMEM((3, 8, 8, 128), jnp.float32),
        pltpu.VMEM((8, 8, 128), jnp.float32),
        pltpu.VMEM((4, 128), jnp.float32),
        pltpu.VMEM((4, 128), jnp.float32),
        pltpu.VMEM((16,), jnp.float32),
        pltpu.VMEM((64,), jnp.float32),
        pltpu.VMEM((H * PITCH,), jnp.float32),
        pltpu.VMEM((8 * PITCH,), jnp.int32),
        pltpu.SemaphoreType.DMA,
        pltpu.SemaphoreType.DMA,
    ]
    return pl.kernel(
        _body,
        out_type=jax.ShapeDtypeStruct((NW * 64,), jnp.float32),
        mesh=mesh,
        scratch_types=scratch,
        compiler_params=pltpu.CompilerParams(needs_layout_passes=False),
    )


_sc_call = _make_call()


def kernel(pred_q, pred_l, weights, gt_q, pred_x, pred_var, gt_t, k):
    km = (jnp.arange(16) < jnp.minimum(jnp.asarray(k, jnp.int32), 8)).astype(jnp.float32)
    qP = pred_q.reshape(128, 128, 64, 4).transpose(2, 0, 3, 1)
    lP = pred_l.reshape(128, 128, 8, 8, 3).transpose(4, 2, 0, 3, 1)
    xP = pred_x.reshape(128, 128, 8, 8, 3).transpose(4, 2, 0, 3, 1)
    vP = pred_var.reshape(128, 128, 8, 8, 3).transpose(4, 2, 0, 3, 1)
    wP = weights.reshape(128, 128, 8, 8).transpose(2, 0, 3, 1)
    gqP = gt_q.reshape(128, 128, 4).transpose(0, 2, 1)
    gtP = jnp.pad(gt_t, ((0, 0), (0, 1))).reshape(128, 128, 4).transpose(0, 2, 1)
    out = _sc_call(qP, lP, wP, gqP, xP, vP, gtP, km)
    s = jnp.sum(out.reshape(NW, 4, 16), axis=(0, 2))
    loss = s[0] / B
    gloss = 0.5 * s[1] / B
    weight_loss = (s[2] - s[3]) / (B * H)
    return (loss, weight_loss, gloss)
